# baseline (speedup 1.0000x reference)
import functools

import jax
import jax.numpy as jnp
from jax import lax
from jax.experimental import pallas as pl
from jax.experimental.pallas import tpu as pltpu
from jax.experimental.pallas import tpu_sc as plsc

GRID_W = 64
N_NEURONS = 4096
B = 1024
D = 32

_NC = 2
_NS = 16
_NW = _NC * _NS
_RPW = B // _NW


def _dist_top2_body(x_ref, w_ref, i1_ref, i2_ref, q1_ref, q2_ref):
    x = x_ref[:, :]
    w = w_ref[:, :]

    xw = jax.lax.dot_general(
        x, w, (((1,), (1,)), ((), ())), preferred_element_type=jnp.float32,
        precision=jax.lax.Precision.HIGHEST,
    )
    xn = jnp.sum(x * x, axis=1, keepdims=True)
    wn = jnp.sum(w * w, axis=1)
    dist = xn - 2.0 * xw + wn[None, :]

    col = jax.lax.broadcasted_iota(jnp.int32, dist.shape, 1)
    i1 = jnp.argmin(dist, axis=1).astype(jnp.int32)
    d1 = jnp.min(dist, axis=1)
    masked = jnp.where(col == i1[:, None], jnp.inf, dist)
    i2 = jnp.argmin(masked, axis=1).astype(jnp.int32)
    d2 = jnp.min(masked, axis=1)

    i1_ref[:, 0] = i1
    i2_ref[:, 0] = i2
    q1_ref[:, 0] = jnp.sqrt(jnp.maximum(d1, 0.0))
    q2_ref[:, 0] = jnp.sqrt(jnp.maximum(d2, 0.0))


_sc_mesh = plsc.VectorSubcoreMesh(
    core_axis_name="c", subcore_axis_name="s", num_cores=_NC, num_subcores=_NS
)


@functools.partial(
    pl.kernel,
    out_type=(
        jax.ShapeDtypeStruct((B, D), jnp.float32),
        jax.ShapeDtypeStruct((B, D), jnp.float32),
    ),
    mesh=_sc_mesh,
    compiler_params=pltpu.CompilerParams(use_tc_tiling_on_sc=False),
    scratch_types=[
        pltpu.VMEM((_RPW,), jnp.int32),
        pltpu.VMEM((_RPW,), jnp.int32),
        pltpu.VMEM((_RPW, D), jnp.float32),
        pltpu.VMEM((_RPW, D), jnp.float32),
        pltpu.SemaphoreType.DMA,
    ],
)
def _sc_gather(w_hbm, i1_hbm, i2_hbm, w1_hbm, w2_hbm,
               i1_v, i2_v, w1_v, w2_v, sem):
    wid = lax.axis_index("s") * _NC + lax.axis_index("c")
    base = wid * _RPW
    pltpu.sync_copy(i1_hbm.at[pl.ds(base, _RPW)], i1_v)
    pltpu.sync_copy(i2_hbm.at[pl.ds(base, _RPW)], i2_v)
    cp1 = pltpu.async_copy(w_hbm.at[i1_v], w1_v, sem)
    cp2 = pltpu.async_copy(w_hbm.at[i2_v], w2_v, sem)
    cp1.wait()
    cp2.wait()
    pltpu.sync_copy(w1_v, w1_hbm.at[pl.ds(base, _RPW)])
    pltpu.sync_copy(w2_v, w2_hbm.at[pl.ds(base, _RPW)])


def _finish_body(x_ref, w1_ref, w2_ref, i1_ref, i2_ref, q1_ref, q2_ref,
                 rc_ref, qe_ref):
    x = x_ref[:, :]
    dd1 = x - w1_ref[:, :]
    dd2 = x - w2_ref[:, :]
    e1 = jnp.sum(dd1 * dd1, axis=1)
    e2 = jnp.sum(dd2 * dd2, axis=1)
    i1 = i1_ref[:, 0]
    i2 = i2_ref[:, 0]
    use2 = (e2 < e1) | ((e2 == e1) & (i2 < i1))
    bmu = jnp.where(use2, i2, i1)
    rc_ref[:, 0] = bmu // GRID_W
    rc_ref[:, 1] = bmu % GRID_W
    qe_ref[:, 0] = jnp.where(use2, q2_ref[:, 0], q1_ref[:, 0])


def kernel(x, weights_map):
    w_flat = jnp.reshape(weights_map, (N_NEURONS, D))
    i1, i2, q1, q2 = pl.pallas_call(
        _dist_top2_body,
        out_shape=(
            jax.ShapeDtypeStruct((B, 1), jnp.int32),
            jax.ShapeDtypeStruct((B, 1), jnp.int32),
            jax.ShapeDtypeStruct((B, 1), jnp.float32),
            jax.ShapeDtypeStruct((B, 1), jnp.float32),
        ),
    )(x, w_flat)
    w1, w2 = _sc_gather(w_flat, jnp.reshape(i1, (B,)), jnp.reshape(i2, (B,)))
    rc, qe = pl.pallas_call(
        _finish_body,
        out_shape=(
            jax.ShapeDtypeStruct((B, 2), jnp.int32),
            jax.ShapeDtypeStruct((B, 1), jnp.float32),
        ),
    )(x, w1, w2, i1, i2, q1, q2)
    return rc, qe[:, 0]


# speedup vs baseline: 2.1859x; 2.1859x over previous
import jax
import jax.numpy as jnp
from jax.experimental import pallas as pl

GRID_W = 64
N_NEURONS = 4096
B = 1024
D = 32


def _som_body(x_ref, w_ref, rc_ref, qe_ref):
    x = x_ref[:, :]
    w = w_ref[:, :]
    xw = jax.lax.dot_general(
        x, w, (((1,), (1,)), ((), ())), preferred_element_type=jnp.float32,
        precision=jax.lax.Precision.HIGHEST,
    )
    xn = jnp.sum(x * x, axis=1, keepdims=True)
    wn = jnp.sum(w * w, axis=1)
    dist = xn - 2.0 * xw + wn[None, :]
    i1 = jnp.argmin(dist, axis=1).astype(jnp.int32)
    d1 = jnp.min(dist, axis=1)
    rc_ref[:, 0] = i1 // GRID_W
    rc_ref[:, 1] = i1 % GRID_W
    qe_ref[:, 0] = jnp.sqrt(jnp.maximum(d1, 0.0))


def kernel(x, weights_map):
    w_flat = jnp.reshape(weights_map, (N_NEURONS, D))
    rc, qe = pl.pallas_call(
        _som_body,
        out_shape=(
            jax.ShapeDtypeStruct((B, 2), jnp.int32),
            jax.ShapeDtypeStruct((B, 1), jnp.float32),
        ),
    )(x, w_flat)
    return rc, qe[:, 0]
